# Initial kernel scaffold; baseline (speedup 1.0000x reference)
import jax
import jax.numpy as jnp
from jax.experimental import pallas as pl


def kernel(x, edge_index, W1, b1, W2, b2):
    raise NotImplementedError("write your pallas kernel here")


# speedup vs baseline: 10.1754x; 10.1754x over previous
import functools

import jax
import jax.numpy as jnp
from jax import lax
from jax.experimental import pallas as pl
from jax.experimental.pallas import tpu as pltpu
from jax.experimental.pallas import tpu_sc as plsc

N_NODES = 10000
D_FEAT = 128
N_CLASSES = 16

NC = 2
NS = 16
NW = NC * NS
LANES = 16

NP = 10240
CHUNK = 128
PAD_NODE = N_NODES

ROWS_PER_TILE = NP // NS


def _edge_pad(n_edges):
    per = NW * CHUNK
    return ((n_edges + per - 1) // per) * per


def _make_deg_kernel(e_pad):
    ept = e_pad // NW
    n_chunks = ept // CHUNK
    mesh = plsc.VectorSubcoreMesh(
        core_axis_name="c", subcore_axis_name="s", num_cores=NC, num_subcores=NS)

    @functools.partial(
        pl.kernel,
        out_type=jax.ShapeDtypeStruct((NC, NP, 16), jnp.float32),
        mesh=mesh,
        scratch_types=[
            pltpu.VMEM((CHUNK,), jnp.int32),
            pltpu.VMEM((CHUNK, 16), jnp.float32),
            pltpu.VMEM_SHARED((NP, 16), jnp.float32),
        ],
    )
    def deg_kernel(dst_hbm, zeros_hbm, parts_hbm, dst_v, ones_v, acc):
        c = lax.axis_index("c")
        s = lax.axis_index("s")
        wid = s * NC + c
        base = wid * ept

        def fill(j, carry):
            ones_v[j, :] = jnp.full((16,), 1.0, jnp.float32)
            return carry
        lax.fori_loop(0, CHUNK, fill, 0)

        @pl.when(s == 0)
        def _():
            pltpu.sync_copy(zeros_hbm, acc)
        plsc.subcore_barrier()

        def body(j, carry):
            pltpu.sync_copy(dst_hbm.at[pl.ds(base + j * CHUNK, CHUNK)], dst_v)
            pltpu.sync_copy(ones_v, acc.at[dst_v], add=True)
            return carry
        lax.fori_loop(0, n_chunks, body, 0)

        plsc.subcore_barrier()
        pltpu.sync_copy(acc.at[pl.ds(s * ROWS_PER_TILE, ROWS_PER_TILE)],
                        parts_hbm.at[c, pl.ds(s * ROWS_PER_TILE, ROWS_PER_TILE)])

    return deg_kernel


def _make_agg_kernel(e_pad, d):
    ept = e_pad // NW
    n_chunks = ept // CHUNK
    mesh = plsc.VectorSubcoreMesh(
        core_axis_name="c", subcore_axis_name="s", num_cores=NC, num_subcores=NS)

    @functools.partial(
        pl.kernel,
        out_type=jax.ShapeDtypeStruct((NC, NP, d), jnp.float32),
        mesh=mesh,
        scratch_types=[
            pltpu.VMEM((CHUNK,), jnp.int32),
            pltpu.VMEM((CHUNK,), jnp.int32),
            pltpu.VMEM((CHUNK, d), jnp.float32),
            pltpu.VMEM_SHARED((NP, d), jnp.float32),
            pltpu.SemaphoreType.DMA,
        ],
    )
    def agg_kernel(src_hbm, dst_hbm, g_hbm, zeros_hbm, parts_hbm,
                   src_v, dst_v, rows_v, acc, sem):
        c = lax.axis_index("c")
        s = lax.axis_index("s")
        wid = s * NC + c
        base = wid * ept

        @pl.when(s == 0)
        def _():
            pltpu.sync_copy(zeros_hbm, acc)
        plsc.subcore_barrier()

        def body(j, carry):
            off = base + j * CHUNK
            pltpu.sync_copy(src_hbm.at[pl.ds(off, CHUNK)], src_v)
            pltpu.sync_copy(dst_hbm.at[pl.ds(off, CHUNK)], dst_v)
            pltpu.async_copy(g_hbm.at[src_v], rows_v, sem).wait()
            pltpu.sync_copy(rows_v, acc.at[dst_v], add=True)
            return carry
        lax.fori_loop(0, n_chunks, body, 0)

        plsc.subcore_barrier()
        pltpu.sync_copy(acc.at[pl.ds(s * ROWS_PER_TILE, ROWS_PER_TILE)],
                        parts_hbm.at[c, pl.ds(s * ROWS_PER_TILE, ROWS_PER_TILE)])

    return agg_kernel


BLK = 512
GRID = NP // BLK


def _prep1_body(parts_ref, x_ref, w1_ref, dinv_ref, g1_ref):
    deg = 1.0 + parts_ref[0] + parts_ref[1]
    dinv = lax.rsqrt(deg)
    dinv_ref[...] = dinv
    h = jnp.dot(x_ref[...], w1_ref[...], preferred_element_type=jnp.float32)
    g1_ref[...] = dinv[:, 0:1] * h


def _prep2_body(parts_ref, g1_ref, dinv_ref, b1_ref, u_ref):
    d1 = dinv_ref[:, 0:1]
    x2 = jax.nn.relu(d1 * (parts_ref[0] + parts_ref[1] + g1_ref[...])
                     + b1_ref[...])
    u_ref[...] = d1 * x2


def _final_body(parts_ref, u_ref, dinv_ref, b2_ref, w2_ref, out_ref):
    t = parts_ref[0] + parts_ref[1] + u_ref[...]
    h2 = jnp.dot(t, w2_ref[...], preferred_element_type=jnp.float32)
    z = jax.nn.relu(dinv_ref[:, 0:1] * h2 + b2_ref[...])
    m = jnp.max(z, axis=1, keepdims=True)
    zs = z - m
    out_ref[...] = zs - jnp.log(jnp.sum(jnp.exp(zs), axis=1, keepdims=True))


def _prep1(deg_parts, x_pad, w1):
    return pl.pallas_call(
        _prep1_body,
        grid=(GRID,),
        in_specs=[
            pl.BlockSpec((NC, BLK, 16), lambda i: (0, i, 0)),
            pl.BlockSpec((BLK, D_FEAT), lambda i: (i, 0)),
            pl.BlockSpec((D_FEAT, D_FEAT), lambda i: (0, 0)),
        ],
        out_specs=[
            pl.BlockSpec((BLK, 16), lambda i: (i, 0)),
            pl.BlockSpec((BLK, D_FEAT), lambda i: (i, 0)),
        ],
        out_shape=[
            jax.ShapeDtypeStruct((NP, 16), jnp.float32),
            jax.ShapeDtypeStruct((NP, D_FEAT), jnp.float32),
        ],
    )(deg_parts, x_pad, w1)


def _prep2(parts1, g1, dinv, b1):
    return pl.pallas_call(
        _prep2_body,
        grid=(GRID,),
        in_specs=[
            pl.BlockSpec((NC, BLK, D_FEAT), lambda i: (0, i, 0)),
            pl.BlockSpec((BLK, D_FEAT), lambda i: (i, 0)),
            pl.BlockSpec((BLK, 16), lambda i: (i, 0)),
            pl.BlockSpec((1, D_FEAT), lambda i: (0, 0)),
        ],
        out_specs=pl.BlockSpec((BLK, D_FEAT), lambda i: (i, 0)),
        out_shape=jax.ShapeDtypeStruct((NP, D_FEAT), jnp.float32),
    )(parts1, g1, dinv, b1)


def _final(parts2, u, dinv, b2, w2):
    return pl.pallas_call(
        _final_body,
        grid=(GRID,),
        in_specs=[
            pl.BlockSpec((NC, BLK, D_FEAT), lambda i: (0, i, 0)),
            pl.BlockSpec((BLK, D_FEAT), lambda i: (i, 0)),
            pl.BlockSpec((BLK, 16), lambda i: (i, 0)),
            pl.BlockSpec((1, N_CLASSES), lambda i: (0, 0)),
            pl.BlockSpec((D_FEAT, N_CLASSES), lambda i: (0, 0)),
        ],
        out_specs=pl.BlockSpec((BLK, N_CLASSES), lambda i: (i, 0)),
        out_shape=jax.ShapeDtypeStruct((NP, N_CLASSES), jnp.float32),
    )(parts2, u, dinv, b2, w2)


def kernel(x, edge_index, W1, b1, W2, b2):
    n_edges = edge_index.shape[1]
    e_pad = _edge_pad(n_edges)

    src = edge_index[0]
    dst = edge_index[1]
    pad = jnp.full((e_pad - n_edges,), PAD_NODE, jnp.int32)
    src_p = jnp.concatenate([src, pad])
    dst_p = jnp.concatenate([dst, pad])

    x_pad = jnp.pad(x, ((0, NP - x.shape[0]), (0, 0)))
    zeros16 = jnp.zeros((NP, 16), jnp.float32)
    zeros128 = jnp.zeros((NP, D_FEAT), jnp.float32)

    deg_parts = _make_deg_kernel(e_pad)(dst_p, zeros16)
    dinv, g1 = _prep1(deg_parts, x_pad, W1)

    parts1 = _make_agg_kernel(e_pad, D_FEAT)(src_p, dst_p, g1, zeros128)
    u = _prep2(parts1, g1, dinv, b1.reshape(1, -1))

    parts2 = _make_agg_kernel(e_pad, D_FEAT)(src_p, dst_p, u, zeros128)
    y = _final(parts2, u, dinv, b2.reshape(1, -1), W2)

    return y[:N_NODES]
